# Initial kernel scaffold; baseline (speedup 1.0000x reference)
import jax
import jax.numpy as jnp
from jax.experimental import pallas as pl


def kernel(keys, values, query, mem_keys, mem_values, usage, kW, kb, vW, vb):
    raise NotImplementedError("write your pallas kernel here")


# speedup vs baseline: 1.2541x; 1.2541x over previous
import jax
import jax.numpy as jnp
from jax import lax
from jax.experimental import pallas as pl
from jax.experimental.pallas import tpu as pltpu

_HI = lax.Precision.HIGHEST
_DEF = lax.Precision.DEFAULT


def _dot_t(a, b, prec=_DEF):
    return lax.dot_general(a, b, (((1,), (1,)), ((), ())), precision=prec)


def _store_kernel(keys_ref, query_ref, mem_keys_ref, kW_ref, kb_ref,
                  vW_ref, vb_ref, kp_ref, sv_ref, tm_ref):
    k = keys_ref[...]
    kp = jnp.tanh(_dot_t(k, kW_ref[...]) + kb_ref[...])
    kp_ref[...] = kp
    q = query_ref[...]
    sv_ref[...] = jnp.tanh(_dot_t(q, vW_ref[...]) + vb_ref[...])
    sim = _dot_t(kp, mem_keys_ref[...])
    m = jnp.max(sim, axis=1, keepdims=True)
    e = jnp.exp(sim - m)
    w = e / jnp.sum(e, axis=1, keepdims=True)
    mw = jnp.max(w, axis=1, keepdims=True)
    cap_iota = lax.broadcasted_iota(jnp.int32, w.shape, 1)
    idx = jnp.min(jnp.where(w == mw, cap_iota, jnp.int32(2**30)), axis=1)
    tm_ref[0, 0, :] = idx


def _scatter_kernel(tm_ref, kp_ref, sv_ref, mem_keys_ref, mem_values_ref,
                    nk_ref, nv_ref):
    c0 = pl.program_id(0) * nk_ref.shape[0]
    tm = tm_ref[0, :]
    cb = nk_ref.shape[0]
    bsz = tm.shape[0]
    c_iota = lax.broadcasted_iota(jnp.int32, (cb, bsz), 0) + c0
    i_iota = lax.broadcasted_iota(jnp.int32, (cb, bsz), 1)
    eq = c_iota == tm[None, :]
    winner = jnp.max(jnp.where(eq, i_iota, -1), axis=1)
    touched = winner >= 0
    onehot = (i_iota == winner[:, None]).astype(jnp.float32)
    nk = lax.dot_general(onehot, kp_ref[...], (((1,), (0,)), ((), ())),
                         precision=_HI)
    nv = lax.dot_general(onehot, sv_ref[...], (((1,), (0,)), ((), ())),
                         precision=_HI)
    nk_ref[...] = jnp.where(touched[:, None], nk, mem_keys_ref[...])
    nv_ref[...] = jnp.where(touched[:, None], nv, mem_values_ref[...])


def _retrieve_kernel(query_ref, kW_ref, kb_ref, nk_ref, nv_ref,
                     att_ref, out_ref):
    q = query_ref[...]
    qp = jnp.tanh(_dot_t(q, kW_ref[...]) + kb_ref[...])
    sim2 = _dot_t(qp, nk_ref[...])
    m = jnp.max(sim2, axis=1, keepdims=True)
    e = jnp.exp(sim2 - m)
    s = jnp.sum(e, axis=1, keepdims=True)
    att = e / s
    att_ref[...] = att
    out_ref[...] = lax.dot_general(att, nv_ref[...], (((1,), (0,)), ((), ())),
                                   precision=_DEF)


def kernel(keys, values, query, mem_keys, mem_values, usage, kW, kb, vW, vb):
    del values, usage
    Bn, K = keys.shape
    C, V = mem_values.shape
    kb2 = kb.reshape(1, K)
    vb2 = vb.reshape(1, V)

    RB = 512
    CB = 512
    RB2 = 256

    rep = lambda *bs: pl.BlockSpec(bs, lambda i: tuple(0 for _ in bs))
    par = pltpu.CompilerParams(dimension_semantics=("parallel",))

    kp, sv, tm3 = pl.pallas_call(
        _store_kernel,
        grid=(Bn // RB,),
        in_specs=[
            pl.BlockSpec((RB, K), lambda i: (i, 0)),
            pl.BlockSpec((RB, K), lambda i: (i, 0)),
            rep(C, K),
            rep(K, K), rep(1, K), rep(V, V), rep(1, V),
        ],
        out_specs=[
            pl.BlockSpec((RB, K), lambda i: (i, 0)),
            pl.BlockSpec((RB, V), lambda i: (i, 0)),
            pl.BlockSpec((1, 1, RB), lambda i: (i, 0, 0)),
        ],
        out_shape=[
            jax.ShapeDtypeStruct((Bn, K), jnp.float32),
            jax.ShapeDtypeStruct((Bn, V), jnp.float32),
            jax.ShapeDtypeStruct((Bn // RB, 1, RB), jnp.int32),
        ],
        compiler_params=par,
    )(keys, query, mem_keys, kW, kb2, vW, vb2)

    tm = tm3.reshape(1, Bn)

    new_keys, new_vals = pl.pallas_call(
        _scatter_kernel,
        grid=(C // CB,),
        in_specs=[
            rep(1, Bn),
            rep(Bn, K), rep(Bn, V),
            pl.BlockSpec((CB, K), lambda i: (i, 0)),
            pl.BlockSpec((CB, V), lambda i: (i, 0)),
        ],
        out_specs=[
            pl.BlockSpec((CB, K), lambda i: (i, 0)),
            pl.BlockSpec((CB, V), lambda i: (i, 0)),
        ],
        out_shape=[
            jax.ShapeDtypeStruct((C, K), jnp.float32),
            jax.ShapeDtypeStruct((C, V), jnp.float32),
        ],
        compiler_params=par,
    )(tm, kp, sv, mem_keys, mem_values)

    attention, retrieved = pl.pallas_call(
        _retrieve_kernel,
        grid=(Bn // RB2,),
        in_specs=[
            pl.BlockSpec((RB2, K), lambda i: (i, 0)),
            rep(K, K), rep(1, K),
            rep(C, K), rep(C, V),
        ],
        out_specs=[
            pl.BlockSpec((RB2, C), lambda i: (i, 0)),
            pl.BlockSpec((RB2, V), lambda i: (i, 0)),
        ],
        out_shape=[
            jax.ShapeDtypeStruct((Bn, C), jnp.float32),
            jax.ShapeDtypeStruct((Bn, V), jnp.float32),
        ],
        compiler_params=par,
    )(query, kW, kb2, new_keys, new_vals)
    return retrieved, attention


# speedup vs baseline: 2.3184x; 1.8487x over previous
import functools

import jax
import jax.numpy as jnp
from jax import lax
from jax.experimental import pallas as pl
from jax.experimental.pallas import tpu as pltpu
from jax.experimental.pallas import tpu_sc as plsc

_HI = lax.Precision.HIGHEST
_DEF = lax.Precision.DEFAULT


def _dot_t(a, b, prec=_DEF):
    return lax.dot_general(a, b, (((1,), (1,)), ((), ())), precision=prec)


def _store_kernel(keys_ref, query_ref, mem_keys_ref, kW_ref, kb_ref,
                  vW_ref, vb_ref, kp_ref, sv_ref, tm_ref):
    k = keys_ref[...]
    kp = jnp.tanh(_dot_t(k, kW_ref[...]) + kb_ref[...])
    kp_ref[...] = kp
    q = query_ref[...]
    sv_ref[...] = jnp.tanh(_dot_t(q, vW_ref[...]) + vb_ref[...])
    sim = _dot_t(kp, mem_keys_ref[...])
    m = jnp.max(sim, axis=1, keepdims=True)
    e = jnp.exp(sim - m)
    w = e / jnp.sum(e, axis=1, keepdims=True)
    mw = jnp.max(w, axis=1, keepdims=True)
    cap_iota = lax.broadcasted_iota(jnp.int32, w.shape, 1)
    idx = jnp.min(jnp.where(w == mw, cap_iota, jnp.int32(2**30)), axis=1)
    tm_ref[0, 0, :] = idx


def _take16(x, idx):
    dn = lax.GatherDimensionNumbers(offset_dims=(), collapsed_slice_dims=(0,),
                                    start_index_map=(0,))
    return lax.gather(x, idx[:, None], dn, slice_sizes=(1,),
                      mode=lax.GatherScatterMode.PROMISE_IN_BOUNDS)


def _make_sc_scatter(Bn, C, K, V):
    L = 16
    NW = 32
    SPW = C // NW
    NB = Bn // L
    mesh = plsc.VectorSubcoreMesh(core_axis_name="c", subcore_axis_name="s")

    @functools.partial(
        pl.kernel, mesh=mesh,
        compiler_params=pltpu.CompilerParams(needs_layout_passes=False,
                                             use_tc_tiling_on_sc=False),
        out_type=[jax.ShapeDtypeStruct((C, K), jnp.float32),
                  jax.ShapeDtypeStruct((C, V), jnp.float32)],
        scratch_types=[
            pltpu.VMEM((Bn,), jnp.int32),
            pltpu.VMEM((SPW,), jnp.int32),
            pltpu.VMEM((128,), jnp.int32),
            pltpu.VMEM((128,), jnp.int32),
            pltpu.VMEM((SPW, K), jnp.float32),
            pltpu.VMEM((SPW, V), jnp.float32),
            pltpu.SemaphoreType.DMA,
        ],
    )
    def sc_scatter(tm_hbm, ktab_hbm, vtab_hbm, nk_hbm, nv_hbm,
                   tm_v, win_v, idx_lo, idx_hi, krows, vrows, sem):
        wid = lax.axis_index("s") * 2 + lax.axis_index("c")
        c0 = wid * SPW
        iota = lax.iota(jnp.int32, L)
        nxt = jnp.minimum(iota + 1, L - 1)
        pltpu.sync_copy(tm_hbm, tm_v)
        for j in range(SPW // L):
            win_v[pl.ds(j * L, L)] = jnp.full((L,), -1, jnp.int32)

        def body(r, _):
            tmv = tm_v[pl.ds(r * L, L)]
            key = tmv * Bn + (r * L + iota)
            sk = lax.sort(key, dimension=0)
            grp = jnp.right_shift(sk, 12)
            row = jnp.bitwise_and(sk, Bn - 1)
            grp_next = _take16(grp, nxt)
            last = (grp != grp_next) | (iota == L - 1)
            inr = (grp >= c0) & (grp < c0 + SPW)
            idx = jnp.clip(grp - c0, 0, SPW - 1)
            plsc.store_scatter(win_v, [idx], row, mask=last & inr)
            return 0

        lax.fori_loop(0, NB, body, 0)

        for j in range(SPW // L):
            w = win_v[pl.ds(j * L, L)]
            cvec = c0 + j * L + iota
            gidx = jnp.where(w >= 0, w, Bn + cvec)
            half = idx_lo if j < (SPW // L) // 2 else idx_hi
            half[pl.ds((j * L) % 128, L)] = gidx

        pltpu.async_copy(ktab_hbm.at[idx_lo], krows.at[pl.ds(0, 128)], sem).wait()
        pltpu.async_copy(ktab_hbm.at[idx_hi], krows.at[pl.ds(128, 128)], sem).wait()
        pltpu.async_copy(vtab_hbm.at[idx_lo], vrows.at[pl.ds(0, 128)], sem).wait()
        pltpu.async_copy(vtab_hbm.at[idx_hi], vrows.at[pl.ds(128, 128)], sem).wait()
        pltpu.sync_copy(krows, nk_hbm.at[pl.ds(c0, SPW)])
        pltpu.sync_copy(vrows, nv_hbm.at[pl.ds(c0, SPW)])

    return sc_scatter


def _retrieve_kernel(query_ref, kW_ref, kb_ref, nk_ref, nv_ref,
                     att_ref, out_ref):
    q = query_ref[...]
    qp = jnp.tanh(_dot_t(q, kW_ref[...]) + kb_ref[...])
    sim2 = _dot_t(qp, nk_ref[...])
    e = jnp.exp(sim2)
    s = jnp.sum(e, axis=1, keepdims=True)
    att = e * (1.0 / s)
    att_ref[...] = att
    out_ref[...] = lax.dot_general(att, nv_ref[...], (((1,), (0,)), ((), ())),
                                   precision=_DEF)


def kernel(keys, values, query, mem_keys, mem_values, usage, kW, kb, vW, vb):
    del values, usage
    Bn, K = keys.shape
    C, V = mem_values.shape
    kb2 = kb.reshape(1, K)
    vb2 = vb.reshape(1, V)

    RB = 512
    CB = 512
    RB2 = 256

    rep = lambda *bs: pl.BlockSpec(bs, lambda i: tuple(0 for _ in bs))
    par = pltpu.CompilerParams(dimension_semantics=("parallel",))

    kp, sv, tm3 = pl.pallas_call(
        _store_kernel,
        grid=(Bn // RB,),
        in_specs=[
            pl.BlockSpec((RB, K), lambda i: (i, 0)),
            pl.BlockSpec((RB, K), lambda i: (i, 0)),
            rep(C, K),
            rep(K, K), rep(1, K), rep(V, V), rep(1, V),
        ],
        out_specs=[
            pl.BlockSpec((RB, K), lambda i: (i, 0)),
            pl.BlockSpec((RB, V), lambda i: (i, 0)),
            pl.BlockSpec((1, 1, RB), lambda i: (i, 0, 0)),
        ],
        out_shape=[
            jax.ShapeDtypeStruct((Bn, K), jnp.float32),
            jax.ShapeDtypeStruct((Bn, V), jnp.float32),
            jax.ShapeDtypeStruct((Bn // RB, 1, RB), jnp.int32),
        ],
        compiler_params=par,
    )(keys, query, mem_keys, kW, kb2, vW, vb2)

    tm = tm3.reshape(Bn)
    ktab = jnp.concatenate([kp, mem_keys], axis=0)
    vtab = jnp.concatenate([sv, mem_values], axis=0)
    new_keys, new_vals = _make_sc_scatter(Bn, C, K, V)(tm, ktab, vtab)

    attention, retrieved = pl.pallas_call(
        _retrieve_kernel,
        grid=(Bn // RB2,),
        in_specs=[
            pl.BlockSpec((RB2, K), lambda i: (i, 0)),
            rep(K, K), rep(1, K),
            rep(C, K), rep(C, V),
        ],
        out_specs=[
            pl.BlockSpec((RB2, C), lambda i: (i, 0)),
            pl.BlockSpec((RB2, V), lambda i: (i, 0)),
        ],
        out_shape=[
            jax.ShapeDtypeStruct((Bn, C), jnp.float32),
            jax.ShapeDtypeStruct((Bn, V), jnp.float32),
        ],
        compiler_params=par,
    )(query, kW, kb2, new_keys, new_vals)
    return retrieved, attention


# speedup vs baseline: 2.3974x; 1.0341x over previous
import functools

import jax
import jax.numpy as jnp
from jax import lax
from jax.experimental import pallas as pl
from jax.experimental.pallas import tpu as pltpu
from jax.experimental.pallas import tpu_sc as plsc

_DEF = lax.Precision.DEFAULT


def _dot_t(a, b, prec=_DEF):
    return lax.dot_general(a, b, (((1,), (1,)), ((), ())), precision=prec)


def _store_kernel(keys_ref, query_ref, mem_keys_ref, kW_ref, kb_ref,
                  vW_ref, vb_ref, kp_ref, sv_ref, tm_ref):
    k = keys_ref[...]
    kp = jnp.tanh(_dot_t(k, kW_ref[...]) + kb_ref[...])
    kp_ref[...] = kp
    q = query_ref[...]
    sv_ref[...] = jnp.tanh(_dot_t(q, vW_ref[...]) + vb_ref[...])
    sim = _dot_t(kp, mem_keys_ref[...])
    m = jnp.max(sim, axis=1, keepdims=True)
    e = jnp.exp(sim - m)
    s = jnp.sum(e, axis=1, keepdims=True)
    w = e / s
    mw = 1.0 / s
    cap_iota = lax.broadcasted_iota(jnp.int32, w.shape, 1)
    idx = jnp.min(jnp.where(w == mw, cap_iota, jnp.int32(2**30)), axis=1)
    tm_ref[0, 0, :] = idx


def _take16(x, idx):
    dn = lax.GatherDimensionNumbers(offset_dims=(), collapsed_slice_dims=(0,),
                                    start_index_map=(0,))
    return lax.gather(x, idx[:, None], dn, slice_sizes=(1,),
                      mode=lax.GatherScatterMode.PROMISE_IN_BOUNDS)


def _make_sc_scatter(Bn, C, K, V):
    L = 16
    NW = 32
    SPW = C // NW
    NB = Bn // L
    mesh = plsc.VectorSubcoreMesh(core_axis_name="c", subcore_axis_name="s")

    @functools.partial(
        pl.kernel, mesh=mesh,
        compiler_params=pltpu.CompilerParams(needs_layout_passes=False,
                                             use_tc_tiling_on_sc=False),
        out_type=[jax.ShapeDtypeStruct((C, K), jnp.float32),
                  jax.ShapeDtypeStruct((C, V), jnp.float32)],
        scratch_types=[
            pltpu.VMEM((Bn,), jnp.int32),
            pltpu.VMEM((SPW,), jnp.int32),
            pltpu.VMEM((128,), jnp.int32),
            pltpu.VMEM((128,), jnp.int32),
            pltpu.VMEM((SPW, K), jnp.float32),
            pltpu.VMEM((SPW, V), jnp.float32),
            pltpu.SemaphoreType.DMA,
        ],
    )
    def sc_scatter(tm_hbm, ktab_hbm, vtab_hbm, nk_hbm, nv_hbm,
                   tm_v, win_v, idx_lo, idx_hi, krows, vrows, sem):
        wid = lax.axis_index("s") * 2 + lax.axis_index("c")
        c0 = wid * SPW
        iota = lax.iota(jnp.int32, L)
        nxt = jnp.minimum(iota + 1, L - 1)
        pltpu.sync_copy(tm_hbm, tm_v)
        for j in range(SPW // L):
            win_v[pl.ds(j * L, L)] = jnp.full((L,), -1, jnp.int32)

        def body(r, _):
            tmv = tm_v[pl.ds(r * L, L)]
            key = tmv * Bn + (r * L + iota)
            sk = lax.sort(key, dimension=0)
            grp = jnp.right_shift(sk, 12)
            row = jnp.bitwise_and(sk, Bn - 1)
            grp_next = _take16(grp, nxt)
            last = (grp != grp_next) | (iota == L - 1)
            inr = (grp >= c0) & (grp < c0 + SPW)
            idx = jnp.clip(grp - c0, 0, SPW - 1)
            plsc.store_scatter(win_v, [idx], row, mask=last & inr)
            return 0

        lax.fori_loop(0, NB, body, 0)

        for j in range(SPW // L):
            w = win_v[pl.ds(j * L, L)]
            cvec = c0 + j * L + iota
            gidx = jnp.where(w >= 0, w, Bn + cvec)
            half = idx_lo if j < (SPW // L) // 2 else idx_hi
            half[pl.ds((j * L) % 128, L)] = gidx

        pltpu.async_copy(ktab_hbm.at[idx_lo], krows.at[pl.ds(0, 128)], sem).wait()
        pltpu.async_copy(ktab_hbm.at[idx_hi], krows.at[pl.ds(128, 128)], sem).wait()
        pltpu.async_copy(vtab_hbm.at[idx_lo], vrows.at[pl.ds(0, 128)], sem).wait()
        pltpu.async_copy(vtab_hbm.at[idx_hi], vrows.at[pl.ds(128, 128)], sem).wait()
        pltpu.sync_copy(krows, nk_hbm.at[pl.ds(c0, SPW)])
        pltpu.sync_copy(vrows, nv_hbm.at[pl.ds(c0, SPW)])

    return sc_scatter


def _retrieve_kernel(query_ref, kW_ref, kb_ref, nk_ref, nv_ref,
                     att_ref, out_ref):
    q = query_ref[...]
    qp = jnp.tanh(_dot_t(q, kW_ref[...]) + kb_ref[...])
    sim2 = _dot_t(qp, nk_ref[...])
    e = jnp.exp(sim2)
    s = jnp.sum(e, axis=1, keepdims=True)
    att = e * (1.0 / s)
    att_ref[...] = att
    out_ref[...] = lax.dot_general(att, nv_ref[...], (((1,), (0,)), ((), ())),
                                   precision=_DEF)


def kernel(keys, values, query, mem_keys, mem_values, usage, kW, kb, vW, vb):
    del values, usage
    Bn, K = keys.shape
    C, V = mem_values.shape
    kb2 = kb.reshape(1, K)
    vb2 = vb.reshape(1, V)

    RB = 512
    RB2 = 256

    rep = lambda *bs: pl.BlockSpec(bs, lambda i: tuple(0 for _ in bs))
    par = pltpu.CompilerParams(dimension_semantics=("parallel",))

    kp, sv, tm3 = pl.pallas_call(
        _store_kernel,
        grid=(Bn // RB,),
        in_specs=[
            pl.BlockSpec((RB, K), lambda i: (i, 0)),
            pl.BlockSpec((RB, K), lambda i: (i, 0)),
            rep(C, K),
            rep(K, K), rep(1, K), rep(V, V), rep(1, V),
        ],
        out_specs=[
            pl.BlockSpec((RB, K), lambda i: (i, 0)),
            pl.BlockSpec((RB, V), lambda i: (i, 0)),
            pl.BlockSpec((1, 1, RB), lambda i: (i, 0, 0)),
        ],
        out_shape=[
            jax.ShapeDtypeStruct((Bn, K), jnp.float32),
            jax.ShapeDtypeStruct((Bn, V), jnp.float32),
            jax.ShapeDtypeStruct((Bn // RB, 1, RB), jnp.int32),
        ],
        compiler_params=par,
    )(keys, query, mem_keys, kW, kb2, vW, vb2)

    tm = tm3.reshape(Bn)
    ktab = jnp.concatenate([kp, mem_keys], axis=0)
    vtab = jnp.concatenate([sv, mem_values], axis=0)
    new_keys, new_vals = _make_sc_scatter(Bn, C, K, V)(tm, ktab, vtab)

    attention, retrieved = pl.pallas_call(
        _retrieve_kernel,
        grid=(Bn // RB2,),
        in_specs=[
            pl.BlockSpec((RB2, K), lambda i: (i, 0)),
            rep(K, K), rep(1, K),
            rep(C, K), rep(C, V),
        ],
        out_specs=[
            pl.BlockSpec((RB2, C), lambda i: (i, 0)),
            pl.BlockSpec((RB2, V), lambda i: (i, 0)),
        ],
        out_shape=[
            jax.ShapeDtypeStruct((Bn, C), jnp.float32),
            jax.ShapeDtypeStruct((Bn, V), jnp.float32),
        ],
        compiler_params=par,
    )(query, kW, kb2, new_keys, new_vals)
    return retrieved, attention


# speedup vs baseline: 2.4546x; 1.0238x over previous
import functools

import jax
import jax.numpy as jnp
from jax import lax
from jax.experimental import pallas as pl
from jax.experimental.pallas import tpu as pltpu
from jax.experimental.pallas import tpu_sc as plsc

_DEF = lax.Precision.DEFAULT


def _dot_t(a, b, prec=_DEF):
    return lax.dot_general(a, b, (((1,), (1,)), ((), ())), precision=prec)


def _store_kernel(keys_ref, query_ref, mem_keys_ref, kW_ref, kb_ref,
                  vW_ref, vb_ref, kp_ref, sv_ref, tm_ref):
    k = keys_ref[...]
    kp = jnp.tanh(_dot_t(k, kW_ref[...]) + kb_ref[...])
    kp_ref[...] = kp
    q = query_ref[...]
    sv_ref[...] = jnp.tanh(_dot_t(q, vW_ref[...]) + vb_ref[...])
    sim = _dot_t(kp, mem_keys_ref[...])
    m = jnp.max(sim, axis=1, keepdims=True)
    e = jnp.exp(sim - m)
    s = jnp.sum(e, axis=1, keepdims=True)
    w = e / s
    mw = 1.0 / s
    cap_iota = lax.broadcasted_iota(jnp.int32, w.shape, 1)
    idx = jnp.min(jnp.where(w == mw, cap_iota, jnp.int32(2**30)), axis=1)
    tm_ref[0, 0, :] = idx


def _take16(x, idx):
    dn = lax.GatherDimensionNumbers(offset_dims=(), collapsed_slice_dims=(0,),
                                    start_index_map=(0,))
    return lax.gather(x, idx[:, None], dn, slice_sizes=(1,),
                      mode=lax.GatherScatterMode.PROMISE_IN_BOUNDS)


def _make_sc_scatter(Bn, C, K, V):
    L = 16
    NW = 32
    SPW = C // NW
    NB = Bn // L
    mesh = plsc.VectorSubcoreMesh(core_axis_name="c", subcore_axis_name="s")

    @functools.partial(
        pl.kernel, mesh=mesh,
        compiler_params=pltpu.CompilerParams(needs_layout_passes=False,
                                             use_tc_tiling_on_sc=False),
        out_type=[jax.ShapeDtypeStruct((C, K), jnp.float32),
                  jax.ShapeDtypeStruct((C, V), jnp.float32)],
        scratch_types=[
            pltpu.VMEM((Bn,), jnp.int32),
            pltpu.VMEM((SPW,), jnp.int32),
            pltpu.VMEM((128,), jnp.int32),
            pltpu.VMEM((128,), jnp.int32),
            pltpu.VMEM((SPW, K), jnp.float32),
            pltpu.VMEM((SPW, V), jnp.float32),
            pltpu.SemaphoreType.DMA,
        ],
    )
    def sc_scatter(tm_hbm, ktab_hbm, vtab_hbm, nk_hbm, nv_hbm,
                   tm_v, win_v, idx_lo, idx_hi, krows, vrows, sem):
        wid = lax.axis_index("s") * 2 + lax.axis_index("c")
        c0 = wid * SPW
        iota = lax.iota(jnp.int32, L)
        nxt = jnp.minimum(iota + 1, L - 1)
        pltpu.sync_copy(tm_hbm, tm_v)
        for j in range(SPW // L):
            win_v[pl.ds(j * L, L)] = jnp.full((L,), -1, jnp.int32)

        def body(r, _):
            tmv = tm_v[pl.ds(r * L, L)]
            key = tmv * Bn + (r * L + iota)
            sk = lax.sort(key, dimension=0)
            grp = jnp.right_shift(sk, 12)
            row = jnp.bitwise_and(sk, Bn - 1)
            grp_next = _take16(grp, nxt)
            last = (grp != grp_next) | (iota == L - 1)
            inr = (grp >= c0) & (grp < c0 + SPW)
            idx = jnp.clip(grp - c0, 0, SPW - 1)
            plsc.store_scatter(win_v, [idx], row, mask=last & inr)
            return 0

        lax.fori_loop(0, NB, body, 0)

        for j in range(SPW // L):
            w = win_v[pl.ds(j * L, L)]
            cvec = c0 + j * L + iota
            gidx = jnp.where(w >= 0, w, Bn + cvec)
            half = idx_lo if j < (SPW // L) // 2 else idx_hi
            half[pl.ds((j * L) % 128, L)] = gidx

        pltpu.async_copy(ktab_hbm.at[idx_lo], krows.at[pl.ds(0, 128)], sem).wait()
        pltpu.async_copy(ktab_hbm.at[idx_hi], krows.at[pl.ds(128, 128)], sem).wait()
        pltpu.async_copy(vtab_hbm.at[idx_lo], vrows.at[pl.ds(0, 128)], sem).wait()
        pltpu.async_copy(vtab_hbm.at[idx_hi], vrows.at[pl.ds(128, 128)], sem).wait()
        pltpu.sync_copy(krows, nk_hbm.at[pl.ds(c0, SPW)])
        pltpu.sync_copy(vrows, nv_hbm.at[pl.ds(c0, SPW)])

    return sc_scatter


def _retrieve_kernel(query_ref, kW_ref, kb_ref, nk_ref, nv_ref,
                     att_ref, out_ref):
    q = query_ref[...]
    qp = jnp.tanh(_dot_t(q, kW_ref[...]) + kb_ref[...])
    sim2 = _dot_t(qp, nk_ref[...])
    e = jnp.exp(sim2)
    s = jnp.sum(e, axis=1, keepdims=True)
    att = e * (1.0 / s)
    att_ref[...] = att
    out_ref[...] = lax.dot_general(att, nv_ref[...], (((1,), (0,)), ((), ())),
                                   precision=_DEF)


def kernel(keys, values, query, mem_keys, mem_values, usage, kW, kb, vW, vb):
    del values, usage
    Bn, K = keys.shape
    C, V = mem_values.shape
    kb2 = kb.reshape(1, K)
    vb2 = vb.reshape(1, V)

    RB = 512
    RB2 = 512

    rep = lambda *bs: pl.BlockSpec(bs, lambda i: tuple(0 for _ in bs))
    par = pltpu.CompilerParams(dimension_semantics=("parallel",))

    kp, sv, tm3 = pl.pallas_call(
        _store_kernel,
        grid=(Bn // RB,),
        in_specs=[
            pl.BlockSpec((RB, K), lambda i: (i, 0)),
            pl.BlockSpec((RB, K), lambda i: (i, 0)),
            rep(C, K),
            rep(K, K), rep(1, K), rep(V, V), rep(1, V),
        ],
        out_specs=[
            pl.BlockSpec((RB, K), lambda i: (i, 0)),
            pl.BlockSpec((RB, V), lambda i: (i, 0)),
            pl.BlockSpec((1, 1, RB), lambda i: (i, 0, 0)),
        ],
        out_shape=[
            jax.ShapeDtypeStruct((Bn, K), jnp.float32),
            jax.ShapeDtypeStruct((Bn, V), jnp.float32),
            jax.ShapeDtypeStruct((Bn // RB, 1, RB), jnp.int32),
        ],
        compiler_params=par,
    )(keys, query, mem_keys, kW, kb2, vW, vb2)

    tm = tm3.reshape(Bn)
    ktab = jnp.concatenate([kp, mem_keys], axis=0)
    vtab = jnp.concatenate([sv, mem_values], axis=0)
    new_keys, new_vals = _make_sc_scatter(Bn, C, K, V)(tm, ktab, vtab)

    attention, retrieved = pl.pallas_call(
        _retrieve_kernel,
        grid=(Bn // RB2,),
        in_specs=[
            pl.BlockSpec((RB2, K), lambda i: (i, 0)),
            rep(K, K), rep(1, K),
            rep(C, K), rep(C, V),
        ],
        out_specs=[
            pl.BlockSpec((RB2, C), lambda i: (i, 0)),
            pl.BlockSpec((RB2, V), lambda i: (i, 0)),
        ],
        out_shape=[
            jax.ShapeDtypeStruct((Bn, C), jnp.float32),
            jax.ShapeDtypeStruct((Bn, V), jnp.float32),
        ],
        compiler_params=par,
    )(query, kW, kb2, new_keys, new_vals)
    return retrieved, attention


# speedup vs baseline: 2.4908x; 1.0148x over previous
import functools

import jax
import jax.numpy as jnp
from jax import lax
from jax.experimental import pallas as pl
from jax.experimental.pallas import tpu as pltpu
from jax.experimental.pallas import tpu_sc as plsc

_DEF = lax.Precision.DEFAULT


def _dot_t(a, b, prec=_DEF):
    return lax.dot_general(a, b, (((1,), (1,)), ((), ())), precision=prec)


def _store_kernel(keys_ref, query_ref, mem_keys_ref, mem_values_ref, kW_ref,
                  kb_ref, vW_ref, vb_ref, ktab_ref, vtab_ref, tm_ref):
    i = pl.program_id(0)
    rb = keys_ref.shape[0]
    k = keys_ref[...]
    kp = jnp.tanh(_dot_t(k, kW_ref[...]) + kb_ref[...])
    ktab_ref[pl.ds(0, rb), :] = kp
    ktab_ref[pl.ds(rb, 2 * rb), :] = mem_keys_ref[pl.ds(i * 2 * rb, 2 * rb), :]
    q = query_ref[...]
    vtab_ref[pl.ds(0, rb), :] = jnp.tanh(_dot_t(q, vW_ref[...]) + vb_ref[...])
    vtab_ref[pl.ds(rb, 2 * rb), :] = mem_values_ref[...]
    sim = _dot_t(kp, mem_keys_ref[...])
    m = jnp.max(sim, axis=1, keepdims=True)
    e = jnp.exp(sim - m)
    s = jnp.sum(e, axis=1, keepdims=True)
    w = e / s
    mw = 1.0 / s
    cap_iota = lax.broadcasted_iota(jnp.int32, w.shape, 1)
    idx = jnp.min(jnp.where(w == mw, cap_iota, jnp.int32(2**30)), axis=1)
    tm_ref[0, 0, :] = idx


def _take16(x, idx):
    dn = lax.GatherDimensionNumbers(offset_dims=(), collapsed_slice_dims=(0,),
                                    start_index_map=(0,))
    return lax.gather(x, idx[:, None], dn, slice_sizes=(1,),
                      mode=lax.GatherScatterMode.PROMISE_IN_BOUNDS)


def _make_sc_scatter(Bn, C, K, V, RB):
    L = 16
    NW = 32
    SPW = C // NW
    NB = Bn // L
    mesh = plsc.VectorSubcoreMesh(core_axis_name="c", subcore_axis_name="s")

    @functools.partial(
        pl.kernel, mesh=mesh,
        compiler_params=pltpu.CompilerParams(needs_layout_passes=False,
                                             use_tc_tiling_on_sc=False),
        out_type=[jax.ShapeDtypeStruct((C, K), jnp.float32),
                  jax.ShapeDtypeStruct((C, V), jnp.float32)],
        scratch_types=[
            pltpu.VMEM((Bn,), jnp.int32),
            pltpu.VMEM((SPW,), jnp.int32),
            pltpu.VMEM((128,), jnp.int32),
            pltpu.VMEM((128,), jnp.int32),
            pltpu.VMEM((SPW, K), jnp.float32),
            pltpu.VMEM((SPW, V), jnp.float32),
            pltpu.SemaphoreType.DMA,
        ],
    )
    def sc_scatter(tm_hbm, ktab_hbm, vtab_hbm, nk_hbm, nv_hbm,
                   tm_v, win_v, idx_lo, idx_hi, krows, vrows, sem):
        wid = lax.axis_index("s") * 2 + lax.axis_index("c")
        c0 = wid * SPW
        iota = lax.iota(jnp.int32, L)
        nxt = jnp.minimum(iota + 1, L - 1)
        pltpu.sync_copy(tm_hbm, tm_v)
        for j in range(SPW // L):
            win_v[pl.ds(j * L, L)] = jnp.full((L,), -1, jnp.int32)

        def body(r, _):
            tmv = tm_v[pl.ds(r * L, L)]
            key = tmv * Bn + (r * L + iota)
            sk = lax.sort(key, dimension=0)
            grp = jnp.right_shift(sk, 12)
            row = jnp.bitwise_and(sk, Bn - 1)
            grp_next = _take16(grp, nxt)
            last = (grp != grp_next) | (iota == L - 1)
            inr = (grp >= c0) & (grp < c0 + SPW)
            idx = jnp.clip(grp - c0, 0, SPW - 1)
            plsc.store_scatter(win_v, [idx], row, mask=last & inr)
            return 0

        lax.fori_loop(0, NB, body, 0)

        for j in range(SPW // L):
            w = win_v[pl.ds(j * L, L)]
            ws = jnp.maximum(w, 0)
            widx = (jnp.right_shift(ws, RB.bit_length() - 1) * (3 * RB)
                    + jnp.bitwise_and(ws, RB - 1))
            cvec = c0 + j * L + iota
            midx = (jnp.right_shift(cvec, RB.bit_length()) * (3 * RB) + RB
                    + jnp.bitwise_and(cvec, 2 * RB - 1))
            gidx = jnp.where(w >= 0, widx, midx)
            half = idx_lo if j < (SPW // L) // 2 else idx_hi
            half[pl.ds((j * L) % 128, L)] = gidx

        pltpu.async_copy(ktab_hbm.at[idx_lo], krows.at[pl.ds(0, 128)], sem).wait()
        pltpu.async_copy(ktab_hbm.at[idx_hi], krows.at[pl.ds(128, 128)], sem).wait()
        pltpu.async_copy(vtab_hbm.at[idx_lo], vrows.at[pl.ds(0, 128)], sem).wait()
        pltpu.async_copy(vtab_hbm.at[idx_hi], vrows.at[pl.ds(128, 128)], sem).wait()
        pltpu.sync_copy(krows, nk_hbm.at[pl.ds(c0, SPW)])
        pltpu.sync_copy(vrows, nv_hbm.at[pl.ds(c0, SPW)])

    return sc_scatter


def _retrieve_kernel(query_ref, kW_ref, kb_ref, nk_ref, nv_ref,
                     att_ref, out_ref):
    q = query_ref[...]
    qp = jnp.tanh(_dot_t(q, kW_ref[...]) + kb_ref[...])
    sim2 = _dot_t(qp, nk_ref[...])
    e = jnp.exp(sim2)
    s = jnp.sum(e, axis=1, keepdims=True)
    att = e * (1.0 / s)
    att_ref[...] = att
    out_ref[...] = lax.dot_general(att, nv_ref[...], (((1,), (0,)), ((), ())),
                                   precision=_DEF)


def kernel(keys, values, query, mem_keys, mem_values, usage, kW, kb, vW, vb):
    del values, usage
    Bn, K = keys.shape
    C, V = mem_values.shape
    kb2 = kb.reshape(1, K)
    vb2 = vb.reshape(1, V)

    RB = 512
    RB2 = 512

    rep = lambda *bs: pl.BlockSpec(bs, lambda i: tuple(0 for _ in bs))
    par = pltpu.CompilerParams(dimension_semantics=("parallel",))

    ktab, vtab, tm3 = pl.pallas_call(
        _store_kernel,
        grid=(Bn // RB,),
        in_specs=[
            pl.BlockSpec((RB, K), lambda i: (i, 0)),
            pl.BlockSpec((RB, K), lambda i: (i, 0)),
            rep(C, K),
            pl.BlockSpec((2 * RB, V), lambda i: (i, 0)),
            rep(K, K), rep(1, K), rep(V, V), rep(1, V),
        ],
        out_specs=[
            pl.BlockSpec((3 * RB, K), lambda i: (i, 0)),
            pl.BlockSpec((3 * RB, V), lambda i: (i, 0)),
            pl.BlockSpec((1, 1, RB), lambda i: (i, 0, 0)),
        ],
        out_shape=[
            jax.ShapeDtypeStruct((Bn + C, K), jnp.float32),
            jax.ShapeDtypeStruct((Bn + C, V), jnp.float32),
            jax.ShapeDtypeStruct((Bn // RB, 1, RB), jnp.int32),
        ],
        compiler_params=par,
    )(keys, query, mem_keys, mem_values, kW, kb2, vW, vb2)

    tm = tm3.reshape(Bn)
    new_keys, new_vals = _make_sc_scatter(Bn, C, K, V, RB)(tm, ktab, vtab)

    attention, retrieved = pl.pallas_call(
        _retrieve_kernel,
        grid=(Bn // RB2,),
        in_specs=[
            pl.BlockSpec((RB2, K), lambda i: (i, 0)),
            rep(K, K), rep(1, K),
            rep(C, K), rep(C, V),
        ],
        out_specs=[
            pl.BlockSpec((RB2, C), lambda i: (i, 0)),
            pl.BlockSpec((RB2, V), lambda i: (i, 0)),
        ],
        out_shape=[
            jax.ShapeDtypeStruct((Bn, C), jnp.float32),
            jax.ShapeDtypeStruct((Bn, V), jnp.float32),
        ],
        compiler_params=par,
    )(query, kW, kb2, new_keys, new_vals)
    return retrieved, attention


# speedup vs baseline: 2.8462x; 1.1427x over previous
import functools

import jax
import jax.numpy as jnp
from jax import lax
from jax.experimental import pallas as pl
from jax.experimental.pallas import tpu as pltpu
from jax.experimental.pallas import tpu_sc as plsc

_DEF = lax.Precision.DEFAULT


def _dot_t(a, b, prec=_DEF):
    return lax.dot_general(a, b, (((1,), (1,)), ((), ())), precision=prec)


def _store_kernel(keys_ref, query_ref, mem_keys_ref, mem_values_ref, kW_ref,
                  kb_ref, vW_ref, vb_ref, ktab_ref, vtab_ref, tm_ref):
    i = pl.program_id(0)
    rb = keys_ref.shape[0]
    k = keys_ref[...]
    kp = jnp.tanh(_dot_t(k, kW_ref[...]) + kb_ref[...])
    ktab_ref[pl.ds(0, rb), :] = kp
    ktab_ref[pl.ds(rb, 2 * rb), :] = mem_keys_ref[pl.ds(i * 2 * rb, 2 * rb), :]
    q = query_ref[...]
    vtab_ref[pl.ds(0, rb), :] = jnp.tanh(_dot_t(q, vW_ref[...]) + vb_ref[...])
    vtab_ref[pl.ds(rb, 2 * rb), :] = mem_values_ref[...]
    sim = _dot_t(kp, mem_keys_ref[...])
    m = jnp.max(sim, axis=1, keepdims=True)
    e = jnp.exp(sim - m)
    s = jnp.sum(e, axis=1, keepdims=True)
    w = e / s
    mw = 1.0 / s
    cap_iota = lax.broadcasted_iota(jnp.int32, w.shape, 1)
    idx = jnp.min(jnp.where(w == mw, cap_iota, jnp.int32(2**30)), axis=1)
    tm_ref[0, 0, :] = idx


def _take16(x, idx):
    dn = lax.GatherDimensionNumbers(offset_dims=(), collapsed_slice_dims=(0,),
                                    start_index_map=(0,))
    return lax.gather(x, idx[:, None], dn, slice_sizes=(1,),
                      mode=lax.GatherScatterMode.PROMISE_IN_BOUNDS)


def _make_sc_scatter(Bn, C, K, V, RB):
    L = 16
    NW = 32
    SPW = C // NW
    NB = Bn // L
    mesh = plsc.VectorSubcoreMesh(core_axis_name="c", subcore_axis_name="s")

    @functools.partial(
        pl.kernel, mesh=mesh,
        compiler_params=pltpu.CompilerParams(needs_layout_passes=False,
                                             use_tc_tiling_on_sc=False),
        out_type=[jax.ShapeDtypeStruct((C, K), jnp.float32),
                  jax.ShapeDtypeStruct((C, V), jnp.float32)],
        scratch_types=[
            pltpu.VMEM((Bn,), jnp.int32),
            pltpu.VMEM((SPW,), jnp.int32),
            pltpu.VMEM((128,), jnp.int32),
            pltpu.VMEM((128,), jnp.int32),
            pltpu.VMEM((SPW, K), jnp.float32),
            pltpu.VMEM((SPW, V), jnp.float32),
            pltpu.SemaphoreType.DMA,
        ],
    )
    def sc_scatter(tm_hbm, ktab_hbm, vtab_hbm, nk_hbm, nv_hbm,
                   tm_v, win_v, idx_lo, idx_hi, krows, vrows, sem):
        wid = lax.axis_index("s") * 2 + lax.axis_index("c")
        c0 = wid * SPW
        iota = lax.iota(jnp.int32, L)
        nxt = jnp.minimum(iota + 1, L - 1)
        pltpu.sync_copy(tm_hbm, tm_v)
        for j in range(SPW // L):
            win_v[pl.ds(j * L, L)] = jnp.full((L,), -1, jnp.int32)

        def body(r, _):
            tmv = tm_v[pl.ds(r * L, L)]
            key = tmv * Bn + (r * L + iota)
            sk = lax.sort(key, dimension=0)
            grp = jnp.right_shift(sk, 12)
            row = jnp.bitwise_and(sk, Bn - 1)
            grp_next = _take16(grp, nxt)
            last = (grp != grp_next) | (iota == L - 1)
            inr = (grp >= c0) & (grp < c0 + SPW)
            idx = jnp.clip(grp - c0, 0, SPW - 1)
            plsc.store_scatter(win_v, [idx], row, mask=last & inr)
            return 0

        lax.fori_loop(0, NB, body, 0)

        for j in range(SPW // L):
            w = win_v[pl.ds(j * L, L)]
            ws = jnp.maximum(w, 0)
            widx = (jnp.right_shift(ws, RB.bit_length() - 1) * (3 * RB)
                    + jnp.bitwise_and(ws, RB - 1))
            cvec = c0 + j * L + iota
            midx = (jnp.right_shift(cvec, RB.bit_length()) * (3 * RB) + RB
                    + jnp.bitwise_and(cvec, 2 * RB - 1))
            gidx = jnp.where(w >= 0, widx, midx)
            half = idx_lo if j < (SPW // L) // 2 else idx_hi
            half[pl.ds((j * L) % 128, L)] = gidx

        pltpu.async_copy(ktab_hbm.at[idx_lo], krows.at[pl.ds(0, 128)], sem).wait()
        pltpu.async_copy(ktab_hbm.at[idx_hi], krows.at[pl.ds(128, 128)], sem).wait()
        pltpu.async_copy(vtab_hbm.at[idx_lo], vrows.at[pl.ds(0, 128)], sem).wait()
        pltpu.async_copy(vtab_hbm.at[idx_hi], vrows.at[pl.ds(128, 128)], sem).wait()
        pltpu.sync_copy(krows, nk_hbm.at[pl.ds(c0, SPW)])
        pltpu.sync_copy(vrows, nv_hbm.at[pl.ds(c0, SPW)])

    return sc_scatter


def _retrieve_kernel(query_ref, kW_ref, kb_ref, nk_ref, nv_ref,
                     att_ref, out_ref):
    q = query_ref[...]
    qp = jnp.tanh(_dot_t(q, kW_ref[...]) + kb_ref[...])
    sim2 = _dot_t(qp, nk_ref[...])
    e = jnp.exp(sim2)
    s = jnp.sum(e, axis=1, keepdims=True)
    r = 1.0 / s
    att_ref[...] = e * r
    ev = lax.dot_general(e, nv_ref[...], (((1,), (0,)), ((), ())),
                         precision=_DEF)
    out_ref[...] = ev * r


def kernel(keys, values, query, mem_keys, mem_values, usage, kW, kb, vW, vb):
    del values, usage
    Bn, K = keys.shape
    C, V = mem_values.shape
    kb2 = kb.reshape(1, K)
    vb2 = vb.reshape(1, V)

    RB = 512
    RB2 = 512

    rep = lambda *bs: pl.BlockSpec(bs, lambda i: tuple(0 for _ in bs))
    par = pltpu.CompilerParams(dimension_semantics=("parallel",))

    ktab, vtab, tm3 = pl.pallas_call(
        _store_kernel,
        grid=(Bn // RB,),
        in_specs=[
            pl.BlockSpec((RB, K), lambda i: (i, 0)),
            pl.BlockSpec((RB, K), lambda i: (i, 0)),
            rep(C, K),
            pl.BlockSpec((2 * RB, V), lambda i: (i, 0)),
            rep(K, K), rep(1, K), rep(V, V), rep(1, V),
        ],
        out_specs=[
            pl.BlockSpec((3 * RB, K), lambda i: (i, 0)),
            pl.BlockSpec((3 * RB, V), lambda i: (i, 0)),
            pl.BlockSpec((1, 1, RB), lambda i: (i, 0, 0)),
        ],
        out_shape=[
            jax.ShapeDtypeStruct((Bn + C, K), jnp.float32),
            jax.ShapeDtypeStruct((Bn + C, V), jnp.float32),
            jax.ShapeDtypeStruct((Bn // RB, 1, RB), jnp.int32),
        ],
        compiler_params=par,
    )(keys, query, mem_keys, mem_values, kW, kb2, vW, vb2)

    tm = tm3.reshape(Bn)
    new_keys, new_vals = _make_sc_scatter(Bn, C, K, V, RB)(tm, ktab, vtab)

    attention, retrieved = pl.pallas_call(
        _retrieve_kernel,
        grid=(Bn // RB2,),
        in_specs=[
            pl.BlockSpec((RB2, K), lambda i: (i, 0)),
            rep(K, K), rep(1, K),
            rep(C, K), rep(C, V),
        ],
        out_specs=[
            pl.BlockSpec((RB2, C), lambda i: (i, 0)),
            pl.BlockSpec((RB2, V), lambda i: (i, 0)),
        ],
        out_shape=[
            jax.ShapeDtypeStruct((Bn, C), jnp.float32),
            jax.ShapeDtypeStruct((Bn, V), jnp.float32),
        ],
        compiler_params=par,
    )(query, kW, kb2, new_keys, new_vals)
    return retrieved, attention


# speedup vs baseline: 2.8501x; 1.0014x over previous
import functools

import jax
import jax.numpy as jnp
from jax import lax
from jax.experimental import pallas as pl
from jax.experimental.pallas import tpu as pltpu
from jax.experimental.pallas import tpu_sc as plsc

_DEF = lax.Precision.DEFAULT


def _dot_t(a, b, prec=_DEF):
    return lax.dot_general(a, b, (((1,), (1,)), ((), ())), precision=prec)


def _store_kernel(keys_ref, query_ref, mem_keys_ref, mem_values_ref, kW_ref,
                  kb_ref, vW_ref, vb_ref, ktab_ref, vtab_ref, tm_ref):
    i = pl.program_id(0)
    rb = keys_ref.shape[0]
    k = keys_ref[...]
    kp = jnp.tanh(_dot_t(k, kW_ref[...]) + kb_ref[...])
    ktab_ref[pl.ds(0, rb), :] = kp
    ktab_ref[pl.ds(rb, 2 * rb), :] = mem_keys_ref[pl.ds(i * 2 * rb, 2 * rb), :]
    q = query_ref[...]
    vtab_ref[pl.ds(0, rb), :] = jnp.tanh(_dot_t(q, vW_ref[...]) + vb_ref[...])
    vtab_ref[pl.ds(rb, 2 * rb), :] = mem_values_ref[...]
    sim = _dot_t(kp, mem_keys_ref[...])
    m = jnp.max(sim, axis=1, keepdims=True)
    e = jnp.exp(sim - m)
    s = jnp.sum(e, axis=1, keepdims=True)
    w = e / s
    mw = 1.0 / s
    cap_iota = lax.broadcasted_iota(jnp.int32, w.shape, 1)
    idx = jnp.min(jnp.where(w == mw, cap_iota, jnp.int32(2**30)), axis=1)
    tm_ref[0, 0, :] = idx


def _take16(x, idx):
    dn = lax.GatherDimensionNumbers(offset_dims=(), collapsed_slice_dims=(0,),
                                    start_index_map=(0,))
    return lax.gather(x, idx[:, None], dn, slice_sizes=(1,),
                      mode=lax.GatherScatterMode.PROMISE_IN_BOUNDS)


def _make_sc_scatter(Bn, C, K, V, RB):
    L = 16
    NW = 32
    SPW = C // NW
    NB = Bn // L
    mesh = plsc.VectorSubcoreMesh(core_axis_name="c", subcore_axis_name="s")

    @functools.partial(
        pl.kernel, mesh=mesh,
        compiler_params=pltpu.CompilerParams(needs_layout_passes=False,
                                             use_tc_tiling_on_sc=False),
        out_type=[jax.ShapeDtypeStruct((C, K), jnp.float32),
                  jax.ShapeDtypeStruct((C, V), jnp.float32)],
        scratch_types=[
            pltpu.VMEM((Bn,), jnp.int32),
            pltpu.VMEM((SPW,), jnp.int32),
            pltpu.VMEM((128,), jnp.int32),
            pltpu.VMEM((128,), jnp.int32),
            pltpu.VMEM((SPW, K), jnp.float32),
            pltpu.VMEM((SPW, V), jnp.float32),
            pltpu.SemaphoreType.DMA,
        ],
    )
    def sc_scatter(tm_hbm, ktab_hbm, vtab_hbm, nk_hbm, nv_hbm,
                   tm_v, win_v, idx_lo, idx_hi, krows, vrows, sem):
        wid = lax.axis_index("s") * 2 + lax.axis_index("c")
        c0 = wid * SPW
        iota = lax.iota(jnp.int32, L)
        nxt = jnp.minimum(iota + 1, L - 1)
        pltpu.sync_copy(tm_hbm, tm_v)
        for j in range(SPW // L):
            win_v[pl.ds(j * L, L)] = jnp.full((L,), -1, jnp.int32)

        def body(r, _):
            tmv = tm_v[pl.ds(r * L, L)]
            key = tmv * Bn + (r * L + iota)
            sk = lax.sort(key, dimension=0)
            grp = jnp.right_shift(sk, 12)
            row = jnp.bitwise_and(sk, Bn - 1)
            grp_next = _take16(grp, nxt)
            last = (grp != grp_next) | (iota == L - 1)
            inr = (grp >= c0) & (grp < c0 + SPW)
            idx = jnp.clip(grp - c0, 0, SPW - 1)
            plsc.store_scatter(win_v, [idx], row, mask=last & inr)
            return 0

        lax.fori_loop(0, NB, body, 0, unroll=4)

        for j in range(SPW // L):
            w = win_v[pl.ds(j * L, L)]
            ws = jnp.maximum(w, 0)
            widx = (jnp.right_shift(ws, RB.bit_length() - 1) * (3 * RB)
                    + jnp.bitwise_and(ws, RB - 1))
            cvec = c0 + j * L + iota
            midx = (jnp.right_shift(cvec, RB.bit_length()) * (3 * RB) + RB
                    + jnp.bitwise_and(cvec, 2 * RB - 1))
            gidx = jnp.where(w >= 0, widx, midx)
            half = idx_lo if j < (SPW // L) // 2 else idx_hi
            half[pl.ds((j * L) % 128, L)] = gidx

        pltpu.async_copy(ktab_hbm.at[idx_lo], krows.at[pl.ds(0, 128)], sem).wait()
        pltpu.async_copy(ktab_hbm.at[idx_hi], krows.at[pl.ds(128, 128)], sem).wait()
        pltpu.async_copy(vtab_hbm.at[idx_lo], vrows.at[pl.ds(0, 128)], sem).wait()
        pltpu.async_copy(vtab_hbm.at[idx_hi], vrows.at[pl.ds(128, 128)], sem).wait()
        pltpu.sync_copy(krows, nk_hbm.at[pl.ds(c0, SPW)])
        pltpu.sync_copy(vrows, nv_hbm.at[pl.ds(c0, SPW)])

    return sc_scatter


def _retrieve_kernel(query_ref, kW_ref, kb_ref, nk_ref, nv_ref,
                     att_ref, out_ref):
    q = query_ref[...]
    qp = jnp.tanh(_dot_t(q, kW_ref[...]) + kb_ref[...])
    sim2 = _dot_t(qp, nk_ref[...])
    e = jnp.exp(sim2)
    s = jnp.sum(e, axis=1, keepdims=True)
    r = 1.0 / s
    att_ref[...] = e * r
    ev = lax.dot_general(e, nv_ref[...], (((1,), (0,)), ((), ())),
                         precision=_DEF)
    out_ref[...] = ev * r


def kernel(keys, values, query, mem_keys, mem_values, usage, kW, kb, vW, vb):
    del values, usage
    Bn, K = keys.shape
    C, V = mem_values.shape
    kb2 = kb.reshape(1, K)
    vb2 = vb.reshape(1, V)

    RB = 512
    RB2 = 512

    rep = lambda *bs: pl.BlockSpec(bs, lambda i: tuple(0 for _ in bs))
    par = pltpu.CompilerParams(dimension_semantics=("parallel",))

    ktab, vtab, tm3 = pl.pallas_call(
        _store_kernel,
        grid=(Bn // RB,),
        in_specs=[
            pl.BlockSpec((RB, K), lambda i: (i, 0)),
            pl.BlockSpec((RB, K), lambda i: (i, 0)),
            rep(C, K),
            pl.BlockSpec((2 * RB, V), lambda i: (i, 0)),
            rep(K, K), rep(1, K), rep(V, V), rep(1, V),
        ],
        out_specs=[
            pl.BlockSpec((3 * RB, K), lambda i: (i, 0)),
            pl.BlockSpec((3 * RB, V), lambda i: (i, 0)),
            pl.BlockSpec((1, 1, RB), lambda i: (i, 0, 0)),
        ],
        out_shape=[
            jax.ShapeDtypeStruct((Bn + C, K), jnp.float32),
            jax.ShapeDtypeStruct((Bn + C, V), jnp.float32),
            jax.ShapeDtypeStruct((Bn // RB, 1, RB), jnp.int32),
        ],
        compiler_params=par,
    )(keys, query, mem_keys, mem_values, kW, kb2, vW, vb2)

    tm = tm3.reshape(Bn)
    new_keys, new_vals = _make_sc_scatter(Bn, C, K, V, RB)(tm, ktab, vtab)

    attention, retrieved = pl.pallas_call(
        _retrieve_kernel,
        grid=(Bn // RB2,),
        in_specs=[
            pl.BlockSpec((RB2, K), lambda i: (i, 0)),
            rep(K, K), rep(1, K),
            rep(C, K), rep(C, V),
        ],
        out_specs=[
            pl.BlockSpec((RB2, C), lambda i: (i, 0)),
            pl.BlockSpec((RB2, V), lambda i: (i, 0)),
        ],
        out_shape=[
            jax.ShapeDtypeStruct((Bn, C), jnp.float32),
            jax.ShapeDtypeStruct((Bn, V), jnp.float32),
        ],
        compiler_params=par,
    )(query, kW, kb2, new_keys, new_vals)
    return retrieved, attention
